# baseline (speedup 1.0000x reference)
import functools

import jax
import jax.numpy as jnp
from jax import lax
from jax.experimental import pallas as pl
from jax.experimental.pallas import tpu as pltpu
from jax.experimental.pallas import tpu_sc as plsc

NC = 2
NS = 16
LANES = 16
COLS_PER_BLOCK = 32768
PPS = 8


@functools.partial(jax.jit, static_argnums=(2, 3))
def _table_matvec(tableT, wb, V, D):
    cb = COLS_PER_BLOCK
    grid = (V + cb - 1) // cb

    def mv(t_ref, wb_ref, o_ref):
        w = wb_ref[0:D, :]
        bias = wb_ref[D, 0]
        o_ref[...] = jnp.sum(t_ref[...] * w, axis=0) + bias

    return pl.pallas_call(
        mv,
        grid=(grid,),
        in_specs=[
            pl.BlockSpec((D, cb), lambda i: (0, i)),
            pl.BlockSpec((D + 8, 1), lambda i: (0, 0)),
        ],
        out_specs=pl.BlockSpec((cb,), lambda i: (i,)),
        out_shape=jax.ShapeDtypeStruct((V,), jnp.float32),
        compiler_params=pltpu.CompilerParams(
            dimension_semantics=("parallel",)
        ),
    )(tableT, wb)


@functools.partial(jax.jit, static_argnums=(2, 3))
def _gather_pool(xt, v, B, L):
    NW = NC * NS
    bpw = B // NW
    nch = bpw // LANES
    assert L % PPS == 0
    nslots = L // PPS

    mesh = plsc.VectorSubcoreMesh(
        core_axis_name="c", subcore_axis_name="s", num_cores=NC, num_subcores=NS
    )

    @functools.partial(
        pl.kernel,
        out_type=jax.ShapeDtypeStruct((B,), jnp.float32),
        mesh=mesh,
        compiler_params=pltpu.CompilerParams(
            needs_layout_passes=False, use_tc_tiling_on_sc=False
        ),
        scratch_types=[
            pltpu.VMEM((L, bpw), jnp.int32),
            pltpu.VMEM((2, PPS, bpw), jnp.float32),
            pltpu.VMEM((bpw,), jnp.float32),
            pltpu.SemaphoreType.DMA,
            pltpu.SemaphoreType.DMA,
        ],
    )
    def kfn(xt_hbm, v_hbm, out_hbm, idx_v, gbuf, out_v, sem_a, sem_b):
        cid = lax.axis_index("c")
        sid = lax.axis_index("s")
        wid = sid * NC + cid
        base = wid * bpw
        pltpu.sync_copy(xt_hbm.at[wid], idx_v)
        last = L - 1
        sems = (sem_a, sem_b)

        def fire(slot, p0, sem):
            for j in range(PPS):
                p = jnp.minimum(p0 + j, last)
                pltpu.async_copy(
                    v_hbm.at[idx_v.at[p]], gbuf.at[slot, j], sem
                )

        def drain(slot, p0, sem):
            for j in range(PPS):
                p = jnp.minimum(p0 + j, last)
                pltpu.make_async_copy(
                    v_hbm.at[idx_v.at[p]], gbuf.at[slot, j], sem
                ).wait()

        def absorb(slot, accs):
            accs = list(accs)
            for j in range(PPS):
                for c in range(nch):
                    accs[c] = accs[c] + gbuf[slot, j, pl.ds(LANES * c, LANES)]
            return tuple(accs)

        fire(0, 0, sems[0])
        fire(1, PPS, sems[1])

        def round_body(i, accs):
            p0 = 2 * PPS * i
            drain(0, p0, sems[0])
            accs = absorb(0, accs)
            fire(0, p0 + 2 * PPS, sems[0])
            drain(1, p0 + PPS, sems[1])
            accs = absorb(1, accs)
            fire(1, p0 + 3 * PPS, sems[1])
            return accs

        accs = lax.fori_loop(
            0, nslots // 2, round_body,
            tuple(jnp.zeros((LANES,), jnp.float32) for _ in range(nch)),
        )
        if nslots % 2 == 1:
            drain(0, PPS * (nslots - 1), sems[0])
            accs = absorb(0, accs)
            drain(1, L, sems[1])
        else:
            drain(0, L, sems[0])
            drain(1, L, sems[1])
        for c in range(nch):
            out_v[pl.ds(LANES * c, LANES)] = accs[c]
        pltpu.sync_copy(out_v, out_hbm.at[pl.ds(base, bpw)])

    return kfn(xt, v)


def kernel(x, table, W, b):
    B, L = x.shape
    V, D = table.shape
    NW = NC * NS
    bpw = B // NW
    xt = x.astype(jnp.int32).reshape(NW, bpw, L).transpose(0, 2, 1)
    wb = jnp.concatenate(
        [
            W * (1.0 / L),
            jnp.broadcast_to(b.astype(jnp.float32) * (1.0 / L), (8, 1)),
        ]
    )
    v = _table_matvec(table.T, wb, V, D)
    out = _gather_pool(xt, v, B, L)
    return out[:, None]


# speedup vs baseline: 1.0046x; 1.0046x over previous
import functools

import jax
import jax.numpy as jnp
from jax import lax
from jax.experimental import pallas as pl
from jax.experimental.pallas import tpu as pltpu
from jax.experimental.pallas import tpu_sc as plsc

NC = 2
NS = 16
LANES = 16
COLS_PER_BLOCK = 33792
PPS = 8


@functools.partial(jax.jit, static_argnums=(2, 3))
def _table_matvec(tableT, wb, V, D):
    cb = COLS_PER_BLOCK
    grid = (V + cb - 1) // cb

    def mv(t_ref, wb_ref, o_ref):
        w = wb_ref[0:D, :]
        bias = wb_ref[D, 0]
        o_ref[...] = jnp.sum(t_ref[...] * w, axis=0) + bias

    return pl.pallas_call(
        mv,
        grid=(grid,),
        in_specs=[
            pl.BlockSpec((D, cb), lambda i: (0, i)),
            pl.BlockSpec((D + 8, 1), lambda i: (0, 0)),
        ],
        out_specs=pl.BlockSpec((cb,), lambda i: (i,)),
        out_shape=jax.ShapeDtypeStruct((V,), jnp.float32),
        compiler_params=pltpu.CompilerParams(
            dimension_semantics=("arbitrary",)
        ),
    )(tableT, wb)


@functools.partial(jax.jit, static_argnums=(2, 3))
def _gather_pool(xt, v, B, L):
    NW = NC * NS
    bpw = B // NW
    nch = bpw // LANES
    assert L % PPS == 0
    nslots = L // PPS

    mesh = plsc.VectorSubcoreMesh(
        core_axis_name="c", subcore_axis_name="s", num_cores=NC, num_subcores=NS
    )

    @functools.partial(
        pl.kernel,
        out_type=jax.ShapeDtypeStruct((B,), jnp.float32),
        mesh=mesh,
        compiler_params=pltpu.CompilerParams(
            needs_layout_passes=False, use_tc_tiling_on_sc=False
        ),
        scratch_types=[
            pltpu.VMEM((L, bpw), jnp.int32),
            pltpu.VMEM((2, PPS, bpw), jnp.float32),
            pltpu.VMEM((bpw,), jnp.float32),
            pltpu.SemaphoreType.DMA,
            pltpu.SemaphoreType.DMA,
        ],
    )
    def kfn(xt_hbm, v_hbm, out_hbm, idx_v, gbuf, out_v, sem_a, sem_b):
        cid = lax.axis_index("c")
        sid = lax.axis_index("s")
        wid = sid * NC + cid
        base = wid * bpw
        pltpu.sync_copy(xt_hbm.at[wid], idx_v)
        last = L - 1
        sems = (sem_a, sem_b)

        def fire(slot, p0, sem):
            for j in range(PPS):
                p = jnp.minimum(p0 + j, last)
                pltpu.async_copy(
                    v_hbm.at[idx_v.at[p]], gbuf.at[slot, j], sem
                )

        def drain(slot, p0, sem):
            for j in range(PPS):
                p = jnp.minimum(p0 + j, last)
                pltpu.make_async_copy(
                    v_hbm.at[idx_v.at[p]], gbuf.at[slot, j], sem
                ).wait()

        def absorb(slot, accs):
            accs = list(accs)
            for j in range(PPS):
                for c in range(nch):
                    accs[c] = accs[c] + gbuf[slot, j, pl.ds(LANES * c, LANES)]
            return tuple(accs)

        fire(0, 0, sems[0])
        fire(1, PPS, sems[1])

        def round_body(i, accs):
            p0 = 2 * PPS * i
            drain(0, p0, sems[0])
            accs = absorb(0, accs)
            fire(0, p0 + 2 * PPS, sems[0])
            drain(1, p0 + PPS, sems[1])
            accs = absorb(1, accs)
            fire(1, p0 + 3 * PPS, sems[1])
            return accs

        accs = lax.fori_loop(
            0, nslots // 2, round_body,
            tuple(jnp.zeros((LANES,), jnp.float32) for _ in range(nch)),
        )
        if nslots % 2 == 1:
            drain(0, PPS * (nslots - 1), sems[0])
            accs = absorb(0, accs)
            drain(1, L, sems[1])
        else:
            drain(0, L, sems[0])
            drain(1, L, sems[1])
        for c in range(nch):
            out_v[pl.ds(LANES * c, LANES)] = accs[c]
        pltpu.sync_copy(out_v, out_hbm.at[pl.ds(base, bpw)])

    return kfn(xt, v)


def kernel(x, table, W, b):
    B, L = x.shape
    V, D = table.shape
    NW = NC * NS
    bpw = B // NW
    xt = x.astype(jnp.int32).reshape(NW, bpw, L).transpose(0, 2, 1)
    wb = jnp.concatenate(
        [
            W * (1.0 / L),
            jnp.broadcast_to(b.astype(jnp.float32) * (1.0 / L), (8, 1)),
        ]
    )
    v = _table_matvec(table.T, wb, V, D)
    out = _gather_pool(xt, v, B, L)
    return out[:, None]
